# Initial kernel scaffold; baseline (speedup 1.0000x reference)
import jax
import jax.numpy as jnp
from jax.experimental import pallas as pl


def kernel(x, params):
    raise NotImplementedError("write your pallas kernel here")


# speedup vs baseline: 1.7990x; 1.7990x over previous
import jax
import jax.numpy as jnp
from jax.experimental import pallas as pl
from jax.experimental.pallas import tpu as pltpu

B, C, T = 64, 64, 2048
EMB = 64
TK = 128
POOL = 4
TOPK = 8
GH = 128
HEADS = 8
NC = 2
D = 2

_F32 = jnp.float32


def _elu(v):
    p = v * (1.0 + v * (0.5 + v * (1.0 / 6.0 + v * (1.0 / 24.0 + v * (1.0 / 120.0)))))
    em1 = jnp.where(v > -0.25, p, jnp.exp(v) - 1.0)
    return jnp.where(v > 0, v, em1)


def _body(
    x_ref,
    dww_ref, s1_ref, b1_ref,
    cse_ref, bse_ref, cso_ref, bso_ref,
    sdwe_ref, sdwo_ref,
    sp2we_ref, sp2wo_ref, s2_ref, b2_ref,
    a3_ref, b3_ref,
    pm_ref,
    wl1_ref, bl1_ref, wr1_ref, br1_ref, we1_ref, att1_ref, gb1_ref,
    wl2_ref, bl2_ref, wr2_ref, br2_ref, we2_ref, att2_ref, gb2_ref,
    fc1w_ref, fc1b_ref, fc2w_ref, fc2b_ref,
    p1w_ref, p1b_ref, p2w_ref, p2b_ref, a1p_ref, a2p_ref,
    lng_ref, lnb_ref,
    c1w_ref, c1b_ref, c2w_ref, c2b_ref, c3w_ref, c3b_ref,
    o_ref,
):
    f32 = _F32
    xb = x_ref[0]

    zpad = jnp.zeros((C, TK // 2), f32)
    xpad = jnp.concatenate([zpad, xb, zpad[:, : TK // 2 - 1]], axis=1)
    dww = dww_ref[...]
    acc = xpad[:, 0:T] * dww[:, 0:1]
    for k in range(1, TK):
        acc = acc + xpad[:, k : k + T] * dww[:, k : k + 1]
    y1 = _elu(acc * s1_ref[...] + b1_ref[...])

    ye = _elu(y1 * cse_ref[...] + bse_ref[...])
    yo = _elu(y1 * cso_ref[...] + bso_ref[...])
    pm = pm_ref[...]
    pe = jnp.concatenate(
        [jnp.dot(ye[:, 512 * m : 512 * (m + 1)], pm, preferred_element_type=f32)
         for m in range(4)], axis=1)
    po = jnp.concatenate(
        [jnp.dot(yo[:, 512 * m : 512 * (m + 1)], pm, preferred_element_type=f32)
         for m in range(4)], axis=1)

    TP = T // POOL
    z8 = jnp.zeros((C, 8), f32)
    pep = jnp.concatenate([z8, pe, z8[:, :7]], axis=1)
    pop = jnp.concatenate([z8, po, z8[:, :7]], axis=1)
    sdwe = sdwe_ref[...]
    sdwo = sdwo_ref[...]
    se = pep[:, 0:TP] * sdwe[:, 0:1]
    so = pop[:, 0:TP] * sdwo[:, 0:1]
    for k in range(1, 16):
        se = se + pep[:, k : k + TP] * sdwe[:, k : k + 1]
        so = so + pop[:, k : k + TP] * sdwo[:, k : k + 1]

    z0 = jnp.dot(sp2we_ref[...], se, preferred_element_type=f32) + jnp.dot(
        sp2wo_ref[...], so, preferred_element_type=f32
    )
    zn = _elu(z0 * s2_ref[...] + b2_ref[...])
    zz = jnp.dot(zn, pm, preferred_element_type=f32)

    v3 = _elu(zz[:, None, :] * a3_ref[...][None] + b3_ref[...][None])
    nf = jnp.mean(v3, axis=-1)

    mu = jnp.mean(xb, axis=1, keepdims=True)
    xm = xb - mu
    var1 = jnp.sum(xm * xm, axis=1, keepdims=True) / (T - 1)
    xs = xm / (jnp.sqrt(var1) + 1e-8)
    corr = jax.lax.dot_general(
        xs, xs, (((1,), (1,)), ((), ())), preferred_element_type=f32
    ) / (T - 1)
    corr = jnp.clip(corr, -1.0, 1.0)

    iota_i = jax.lax.broadcasted_iota(jnp.int32, (C, C), 0)
    iota_j = jax.lax.broadcasted_iota(jnp.int32, (C, C), 1)
    score = jnp.where(iota_i == iota_j, -1.0, jnp.abs(corr))
    amask = jnp.zeros((C, C), f32)
    for _ in range(TOPK):
        mrow = jnp.max(score, axis=1, keepdims=True)
        cand = jnp.where(score == mrow, iota_j, C)
        jmin = jnp.min(cand, axis=1, keepdims=True)
        hit = iota_j == jmin
        amask = jnp.where(hit, 1.0, amask)
        score = jnp.where(hit, score - 3.0, score)
    atmask = jnp.swapaxes(amask, 0, 1)

    xl = jnp.dot(nf, wl1_ref[...], preferred_element_type=f32) + bl1_ref[...]
    xr = jnp.dot(nf, wr1_ref[...], preferred_element_type=f32) + br1_ref[...]
    we1 = we1_ref[...]
    att1 = att1_ref[...]
    outs = []
    for h in range(HEADS):
        el = xl[:, GH * h : GH * (h + 1)]
        er = xr[:, GH * h : GH * (h + 1)]
        weh = we1[h : h + 1, :][None]
        ath = att1[h : h + 1, :][None]
        t3 = er[:, None, :] + el[None, :, :] + corr[:, :, None] * weh
        t3 = jnp.where(t3 >= 0, t3, 0.2 * t3) * ath
        et = jnp.sum(t3, axis=-1)
        em = jnp.where(atmask > 0.5, et, -1e30)
        mj = jnp.max(em, axis=1, keepdims=True)
        mj = jnp.where(mj > -1e29, mj, 0.0)
        ex = jnp.where(atmask > 0.5, jnp.exp(et - mj), 0.0)
        sj = jnp.sum(ex, axis=1, keepdims=True)
        alpha = ex / (sj + 1e-16)
        outs.append(jnp.dot(alpha, el, preferred_element_type=f32))
    h1 = _elu(jnp.concatenate(outs, axis=1) + gb1_ref[...])

    xl2 = jnp.dot(h1, wl2_ref[...], preferred_element_type=f32) + bl2_ref[...]
    xr2 = jnp.dot(h1, wr2_ref[...], preferred_element_type=f32) + br2_ref[...]
    t3 = xr2[:, None, :] + xl2[None, :, :] + corr[:, :, None] * we2_ref[...][None]
    t3 = jnp.where(t3 >= 0, t3, 0.2 * t3) * att2_ref[...][None]
    et = jnp.sum(t3, axis=-1)
    em = jnp.where(atmask > 0.5, et, -1e30)
    mj = jnp.max(em, axis=1, keepdims=True)
    mj = jnp.where(mj > -1e29, mj, 0.0)
    ex = jnp.where(atmask > 0.5, jnp.exp(et - mj), 0.0)
    sj = jnp.sum(ex, axis=1, keepdims=True)
    alpha = ex / (sj + 1e-16)
    h2 = _elu(jnp.dot(alpha, xl2, preferred_element_type=f32) + gb2_ref[...])

    g = jnp.sum(h2, axis=0, keepdims=True) / C
    a1v = jnp.tanh(jnp.dot(g, fc1w_ref[...], preferred_element_type=f32) + fc1b_ref[...])
    a2v = jnp.tanh(jnp.dot(g, fc2w_ref[...], preferred_element_type=f32) + fc2b_ref[...])
    gg = (
        g
        + (jnp.dot(a1v, p1w_ref[...], preferred_element_type=f32) + p1b_ref[...]) * a1p_ref[...]
        + (jnp.dot(a2v * a2v, p2w_ref[...], preferred_element_type=f32) + p2b_ref[...]) * a2p_ref[...]
    )
    gmu = jnp.mean(gg, axis=1, keepdims=True)
    gvar = jnp.mean((gg - gmu) * (gg - gmu), axis=1, keepdims=True)
    gn = (gg - gmu) / jnp.sqrt(gvar + 1e-5) * lng_ref[...] + lnb_ref[...]
    hh1 = jax.nn.relu(jnp.dot(gn, c1w_ref[...], preferred_element_type=f32) + c1b_ref[...])
    hh2 = jax.nn.relu(jnp.dot(hh1, c2w_ref[...], preferred_element_type=f32) + c2b_ref[...])
    res = jnp.dot(hh2, c3w_ref[...], preferred_element_type=f32) + c3b_ref[...]
    o_ref[...] = res[None]


def kernel(x, params):
    p = params
    f32 = _F32
    inv = 1.0 / jnp.sqrt(jnp.asarray(1.0 + 1e-5, f32))

    col = lambda v: v.reshape(-1, 1).astype(f32)
    row = lambda v: v.reshape(1, -1).astype(f32)

    dww = p['dw_w'].reshape(C, TK)
    s1 = col(p['bn1_g'] * inv)
    b1 = col(p['bn1_b'])
    spw = p['sp_w'].reshape(C * D)
    ss = p['bns_g'] * inv
    cse = col(spw[0::2] * ss[0::2])
    bse = col(p['bns_b'][0::2])
    cso = col(spw[1::2] * ss[1::2])
    bso = col(p['bns_b'][1::2])
    sdw = p['sd_w'].reshape(C * D, 16)
    sdwe = sdw[0::2]
    sdwo = sdw[1::2]
    sp2 = p['sp2_w'].reshape(C, C * D)
    sp2we = sp2[:, 0::2]
    sp2wo = sp2[:, 1::2]
    s2 = col(p['bn2_g'] * inv)
    b2 = col(p['bn2_b'])
    a3 = col(p['pw_w'].reshape(EMB) * (p['bn3_g'] * inv))
    b3 = col(p['bn3_b'])

    poolm = (jnp.arange(512)[:, None] // POOL == jnp.arange(128)[None, :]).astype(f32) / POOL

    wl1 = p['g1_Wl'].T
    wr1 = p['g1_Wr'].T
    we1 = p['g1_We'].reshape(HEADS, GH)
    att1 = p['g1_att'].reshape(HEADS, GH)
    wl2 = p['g2_Wl'].T
    wr2 = p['g2_Wr'].T
    we2 = p['g2_We'].reshape(1, GH)
    att2 = p['g2_att'].reshape(1, GH)

    operands = [
        dww, s1, b1,
        cse, bse, cso, bso,
        sdwe, sdwo,
        sp2we, sp2wo, s2, b2,
        a3, b3,
        poolm,
        wl1, row(p['g1_bl']), wr1, row(p['g1_br']), we1, att1, row(p['g1_bias']),
        wl2, row(p['g2_bl']), wr2, row(p['g2_br']), we2, att2, row(p['g2_bias']),
        p['fc1_w'].T, row(p['fc1_b']), p['fc2_w'].T, row(p['fc2_b']),
        p['p1_w'].T, row(p['p1_b']), p['p2_w'].T, row(p['p2_b']),
        row(p['a1']), row(p['a2']),
        row(p['ln_g']), row(p['ln_b']),
        p['c1_w'].T, row(p['c1_b']), p['c2_w'].T, row(p['c2_b']),
        p['c3_w'].T, row(p['c3_b']),
    ]

    wspecs = [
        pl.BlockSpec(o.shape, lambda b, _n=o.ndim: (0,) * _n) for o in operands
    ]
    return pl.pallas_call(
        _body,
        grid=(B,),
        in_specs=[pl.BlockSpec((1, C, T), lambda b: (b, 0, 0))] + wspecs,
        out_specs=pl.BlockSpec((1, 1, NC), lambda b: (b, 0, 0)),
        out_shape=jax.ShapeDtypeStruct((B, 1, NC), f32),
        compiler_params=pltpu.CompilerParams(
            dimension_semantics=("arbitrary",),
            vmem_limit_bytes=100 * 1024 * 1024,
        ),
    )(x.astype(f32), *operands).reshape(B, NC)


# speedup vs baseline: 1.8019x; 1.0016x over previous
import jax
import jax.numpy as jnp
from jax.experimental import pallas as pl
from jax.experimental.pallas import tpu as pltpu

B, C, T = 64, 64, 2048
EMB = 64
TK = 128
POOL = 4
TOPK = 8
GH = 128
HEADS = 8
NC = 2
D = 2

_F32 = jnp.float32


def _elu(v):
    p = v * (1.0 + v * (0.5 + v * (1.0 / 6.0 + v * (1.0 / 24.0 + v * (1.0 / 120.0)))))
    em1 = jnp.where(v > -0.25, p, jnp.exp(v) - 1.0)
    return jnp.where(v > 0, v, em1)


def _body(
    x_ref,
    dww_ref, s1_ref, b1_ref,
    cse_ref, bse_ref, cso_ref, bso_ref,
    sdwe_ref, sdwo_ref,
    sp2we_ref, sp2wo_ref, s2_ref, b2_ref,
    a3_ref, b3_ref,
    pm_ref,
    wl1_ref, bl1_ref, wr1_ref, br1_ref, we1_ref, att1_ref, gb1_ref,
    wl2_ref, bl2_ref, wr2_ref, br2_ref, we2_ref, att2_ref, gb2_ref,
    fc1w_ref, fc1b_ref, fc2w_ref, fc2b_ref,
    p1w_ref, p1b_ref, p2w_ref, p2b_ref, a1p_ref, a2p_ref,
    lng_ref, lnb_ref,
    c1w_ref, c1b_ref, c2w_ref, c2b_ref, c3w_ref, c3b_ref,
    o_ref,
):
    f32 = _F32
    xb = x_ref[0]

    zpad = jnp.zeros((C, TK // 2), f32)
    xpad = jnp.concatenate([zpad, xb, zpad[:, : TK // 2 - 1]], axis=1)
    dww = dww_ref[...]
    acc = xpad[:, 0:T] * dww[:, 0:1]
    for k in range(1, TK):
        acc = acc + xpad[:, k : k + T] * dww[:, k : k + 1]
    y1 = _elu(acc * s1_ref[...] + b1_ref[...])

    ye = _elu(y1 * cse_ref[...] + bse_ref[...])
    yo = _elu(y1 * cso_ref[...] + bso_ref[...])
    pm = pm_ref[...]
    pe = jnp.concatenate(
        [jnp.dot(ye[:, 512 * m : 512 * (m + 1)], pm, preferred_element_type=f32)
         for m in range(4)], axis=1)
    po = jnp.concatenate(
        [jnp.dot(yo[:, 512 * m : 512 * (m + 1)], pm, preferred_element_type=f32)
         for m in range(4)], axis=1)

    TP = T // POOL
    z8 = jnp.zeros((C, 8), f32)
    pep = jnp.concatenate([z8, pe, z8[:, :7]], axis=1)
    pop = jnp.concatenate([z8, po, z8[:, :7]], axis=1)
    sdwe = sdwe_ref[...]
    sdwo = sdwo_ref[...]
    se = pep[:, 0:TP] * sdwe[:, 0:1]
    so = pop[:, 0:TP] * sdwo[:, 0:1]
    for k in range(1, 16):
        se = se + pep[:, k : k + TP] * sdwe[:, k : k + 1]
        so = so + pop[:, k : k + TP] * sdwo[:, k : k + 1]

    z0 = jnp.dot(sp2we_ref[...], se, preferred_element_type=f32) + jnp.dot(
        sp2wo_ref[...], so, preferred_element_type=f32
    )
    zn = _elu(z0 * s2_ref[...] + b2_ref[...])
    zz = jnp.dot(zn, pm, preferred_element_type=f32)

    v3 = _elu(zz[:, None, :] * a3_ref[...][None] + b3_ref[...][None])
    nf = jnp.mean(v3, axis=-1)

    mu = jnp.mean(xb, axis=1, keepdims=True)
    xm = xb - mu
    var1 = jnp.sum(xm * xm, axis=1, keepdims=True) / (T - 1)
    xs = xm / (jnp.sqrt(var1) + 1e-8)
    corr = jax.lax.dot_general(
        xs, xs, (((1,), (1,)), ((), ())), preferred_element_type=f32
    ) / (T - 1)
    corr = jnp.clip(corr, -1.0, 1.0)

    iota_i = jax.lax.broadcasted_iota(jnp.int32, (C, C), 0)
    iota_j = jax.lax.broadcasted_iota(jnp.int32, (C, C), 1)
    score = jnp.where(iota_i == iota_j, -1.0, jnp.abs(corr))
    amask = jnp.zeros((C, C), f32)
    for _ in range(TOPK):
        mrow = jnp.max(score, axis=1, keepdims=True)
        cand = jnp.where(score == mrow, iota_j, C)
        jmin = jnp.min(cand, axis=1, keepdims=True)
        hit = iota_j == jmin
        amask = jnp.where(hit, 1.0, amask)
        score = jnp.where(hit, score - 3.0, score)
    atmask = jnp.swapaxes(amask, 0, 1)

    xl = jnp.dot(nf, wl1_ref[...], preferred_element_type=f32) + bl1_ref[...]
    xr = jnp.dot(nf, wr1_ref[...], preferred_element_type=f32) + br1_ref[...]
    we1 = we1_ref[...]
    att1 = att1_ref[...]
    outs = []
    for h in range(HEADS):
        el = xl[:, GH * h : GH * (h + 1)]
        er = xr[:, GH * h : GH * (h + 1)]
        weh = we1[h : h + 1, :][None]
        ath = att1[h : h + 1, :][None]
        t3 = er[:, None, :] + el[None, :, :] + corr[:, :, None] * weh
        t3 = jnp.where(t3 >= 0, t3, 0.2 * t3) * ath
        et = jnp.sum(t3, axis=-1)
        em = jnp.where(atmask > 0.5, et, -1e30)
        mj = jnp.max(em, axis=1, keepdims=True)
        mj = jnp.where(mj > -1e29, mj, 0.0)
        ex = jnp.where(atmask > 0.5, jnp.exp(et - mj), 0.0)
        sj = jnp.sum(ex, axis=1, keepdims=True)
        alpha = ex / (sj + 1e-16)
        outs.append(jnp.dot(alpha, el, preferred_element_type=f32))
    h1 = _elu(jnp.concatenate(outs, axis=1) + gb1_ref[...])

    xl2 = jnp.dot(h1, wl2_ref[...], preferred_element_type=f32) + bl2_ref[...]
    xr2 = jnp.dot(h1, wr2_ref[...], preferred_element_type=f32) + br2_ref[...]
    t3 = xr2[:, None, :] + xl2[None, :, :] + corr[:, :, None] * we2_ref[...][None]
    t3 = jnp.where(t3 >= 0, t3, 0.2 * t3) * att2_ref[...][None]
    et = jnp.sum(t3, axis=-1)
    em = jnp.where(atmask > 0.5, et, -1e30)
    mj = jnp.max(em, axis=1, keepdims=True)
    mj = jnp.where(mj > -1e29, mj, 0.0)
    ex = jnp.where(atmask > 0.5, jnp.exp(et - mj), 0.0)
    sj = jnp.sum(ex, axis=1, keepdims=True)
    alpha = ex / (sj + 1e-16)
    h2 = _elu(jnp.dot(alpha, xl2, preferred_element_type=f32) + gb2_ref[...])

    g = jnp.sum(h2, axis=0, keepdims=True) / C
    a1v = jnp.tanh(jnp.dot(g, fc1w_ref[...], preferred_element_type=f32) + fc1b_ref[...])
    a2v = jnp.tanh(jnp.dot(g, fc2w_ref[...], preferred_element_type=f32) + fc2b_ref[...])
    gg = (
        g
        + (jnp.dot(a1v, p1w_ref[...], preferred_element_type=f32) + p1b_ref[...]) * a1p_ref[...]
        + (jnp.dot(a2v * a2v, p2w_ref[...], preferred_element_type=f32) + p2b_ref[...]) * a2p_ref[...]
    )
    gmu = jnp.mean(gg, axis=1, keepdims=True)
    gvar = jnp.mean((gg - gmu) * (gg - gmu), axis=1, keepdims=True)
    gn = (gg - gmu) / jnp.sqrt(gvar + 1e-5) * lng_ref[...] + lnb_ref[...]
    hh1 = jax.nn.relu(jnp.dot(gn, c1w_ref[...], preferred_element_type=f32) + c1b_ref[...])
    hh2 = jax.nn.relu(jnp.dot(hh1, c2w_ref[...], preferred_element_type=f32) + c2b_ref[...])
    res = jnp.dot(hh2, c3w_ref[...], preferred_element_type=f32) + c3b_ref[...]
    o_ref[...] = res[None]


def kernel(x, params):
    p = params
    f32 = _F32
    inv = 1.0 / jnp.sqrt(jnp.asarray(1.0 + 1e-5, f32))

    col = lambda v: v.reshape(-1, 1).astype(f32)
    row = lambda v: v.reshape(1, -1).astype(f32)

    dww = p['dw_w'].reshape(C, TK)
    s1 = col(p['bn1_g'] * inv)
    b1 = col(p['bn1_b'])
    spw = p['sp_w'].reshape(C * D)
    ss = p['bns_g'] * inv
    cse = col(spw[0::2] * ss[0::2])
    bse = col(p['bns_b'][0::2])
    cso = col(spw[1::2] * ss[1::2])
    bso = col(p['bns_b'][1::2])
    sdw = p['sd_w'].reshape(C * D, 16)
    sdwe = sdw[0::2]
    sdwo = sdw[1::2]
    sp2 = p['sp2_w'].reshape(C, C * D)
    sp2we = sp2[:, 0::2]
    sp2wo = sp2[:, 1::2]
    s2 = col(p['bn2_g'] * inv)
    b2 = col(p['bn2_b'])
    a3 = col(p['pw_w'].reshape(EMB) * (p['bn3_g'] * inv))
    b3 = col(p['bn3_b'])

    poolm = (jnp.arange(512)[:, None] // POOL == jnp.arange(128)[None, :]).astype(f32) / POOL

    wl1 = p['g1_Wl'].T
    wr1 = p['g1_Wr'].T
    we1 = p['g1_We'].reshape(HEADS, GH)
    att1 = p['g1_att'].reshape(HEADS, GH)
    wl2 = p['g2_Wl'].T
    wr2 = p['g2_Wr'].T
    we2 = p['g2_We'].reshape(1, GH)
    att2 = p['g2_att'].reshape(1, GH)

    operands = [
        dww, s1, b1,
        cse, bse, cso, bso,
        sdwe, sdwo,
        sp2we, sp2wo, s2, b2,
        a3, b3,
        poolm,
        wl1, row(p['g1_bl']), wr1, row(p['g1_br']), we1, att1, row(p['g1_bias']),
        wl2, row(p['g2_bl']), wr2, row(p['g2_br']), we2, att2, row(p['g2_bias']),
        p['fc1_w'].T, row(p['fc1_b']), p['fc2_w'].T, row(p['fc2_b']),
        p['p1_w'].T, row(p['p1_b']), p['p2_w'].T, row(p['p2_b']),
        row(p['a1']), row(p['a2']),
        row(p['ln_g']), row(p['ln_b']),
        p['c1_w'].T, row(p['c1_b']), p['c2_w'].T, row(p['c2_b']),
        p['c3_w'].T, row(p['c3_b']),
    ]

    wspecs = [
        pl.BlockSpec(o.shape, lambda b, _n=o.ndim: (0,) * _n) for o in operands
    ]
    return pl.pallas_call(
        _body,
        grid=(B,),
        in_specs=[pl.BlockSpec((1, C, T), lambda b: (b, 0, 0))] + wspecs,
        out_specs=pl.BlockSpec((1, 1, NC), lambda b: (b, 0, 0)),
        out_shape=jax.ShapeDtypeStruct((B, 1, NC), f32),
        compiler_params=pltpu.CompilerParams(
            dimension_semantics=("parallel",),
            vmem_limit_bytes=100 * 1024 * 1024,
        ),
    )(x.astype(f32), *operands).reshape(B, NC)


# speedup vs baseline: 1.8201x; 1.0101x over previous
import jax
import jax.numpy as jnp
from jax.experimental import pallas as pl
from jax.experimental.pallas import tpu as pltpu

B, C, T = 64, 64, 2048
EMB = 64
TK = 128
POOL = 4
TOPK = 8
GH = 128
HEADS = 8
NC = 2
D = 2

_F32 = jnp.float32


def _elu(v):
    p = v * (1.0 + v * (0.5 + v * (1.0 / 6.0 + v * (1.0 / 24.0 + v * (1.0 / 120.0)))))
    em1 = jnp.where(v > -0.25, p, jnp.exp(v) - 1.0)
    return jnp.where(v > 0, v, em1)


def _body(
    x_ref,
    dww_ref, s1_ref, b1_ref,
    cse_ref, bse_ref, cso_ref, bso_ref,
    sdwe_ref, sdwo_ref,
    sp2we_ref, sp2wo_ref, s2_ref, b2_ref,
    a3_ref, b3_ref,
    pm_ref,
    wl1_ref, bl1_ref, wr1_ref, br1_ref, we1_ref, att1_ref, gb1_ref,
    wl2_ref, bl2_ref, wr2_ref, br2_ref, we2_ref, att2_ref, gb2_ref,
    fc1w_ref, fc1b_ref, fc2w_ref, fc2b_ref,
    p1w_ref, p1b_ref, p2w_ref, p2b_ref, a1p_ref, a2p_ref,
    lng_ref, lnb_ref,
    c1w_ref, c1b_ref, c2w_ref, c2b_ref, c3w_ref, c3b_ref,
    o_ref,
):
    f32 = _F32
    xb = x_ref[0]

    zpad = jnp.zeros((C, TK // 2), f32)
    xpad = jnp.concatenate([zpad, xb, zpad[:, : TK // 2 - 1]], axis=1)
    dww = dww_ref[...]
    accs = [xpad[:, a : a + T] * dww[:, a : a + 1] for a in range(8)]
    for k in range(8, TK):
        a = k % 8
        accs[a] = accs[a] + xpad[:, k : k + T] * dww[:, k : k + 1]
    acc = ((accs[0] + accs[1]) + (accs[2] + accs[3])) + (
        (accs[4] + accs[5]) + (accs[6] + accs[7])
    )
    y1 = _elu(acc * s1_ref[...] + b1_ref[...])

    ye = _elu(y1 * cse_ref[...] + bse_ref[...])
    yo = _elu(y1 * cso_ref[...] + bso_ref[...])
    pm = pm_ref[...]
    pe = jnp.concatenate(
        [jnp.dot(ye[:, 512 * m : 512 * (m + 1)], pm, preferred_element_type=f32)
         for m in range(4)], axis=1)
    po = jnp.concatenate(
        [jnp.dot(yo[:, 512 * m : 512 * (m + 1)], pm, preferred_element_type=f32)
         for m in range(4)], axis=1)

    TP = T // POOL
    z8 = jnp.zeros((C, 8), f32)
    pep = jnp.concatenate([z8, pe, z8[:, :7]], axis=1)
    pop = jnp.concatenate([z8, po, z8[:, :7]], axis=1)
    sdwe = sdwe_ref[...]
    sdwo = sdwo_ref[...]
    ses = [pep[:, a : a + TP] * sdwe[:, a : a + 1] for a in range(4)]
    sos = [pop[:, a : a + TP] * sdwo[:, a : a + 1] for a in range(4)]
    for k in range(4, 16):
        a = k % 4
        ses[a] = ses[a] + pep[:, k : k + TP] * sdwe[:, k : k + 1]
        sos[a] = sos[a] + pop[:, k : k + TP] * sdwo[:, k : k + 1]
    se = (ses[0] + ses[1]) + (ses[2] + ses[3])
    so = (sos[0] + sos[1]) + (sos[2] + sos[3])

    z0 = jnp.dot(sp2we_ref[...], se, preferred_element_type=f32) + jnp.dot(
        sp2wo_ref[...], so, preferred_element_type=f32
    )
    zn = _elu(z0 * s2_ref[...] + b2_ref[...])
    zz = jnp.dot(zn, pm, preferred_element_type=f32)

    v3 = _elu(zz[:, None, :] * a3_ref[...][None] + b3_ref[...][None])
    nf = jnp.mean(v3, axis=-1)

    mu = jnp.mean(xb, axis=1, keepdims=True)
    xm = xb - mu
    var1 = jnp.sum(xm * xm, axis=1, keepdims=True) / (T - 1)
    xs = xm / (jnp.sqrt(var1) + 1e-8)
    corr = jax.lax.dot_general(
        xs, xs, (((1,), (1,)), ((), ())), preferred_element_type=f32
    ) / (T - 1)
    corr = jnp.clip(corr, -1.0, 1.0)

    iota_i = jax.lax.broadcasted_iota(jnp.int32, (C, C), 0)
    iota_j = jax.lax.broadcasted_iota(jnp.int32, (C, C), 1)
    score = jnp.where(iota_i == iota_j, -1.0, jnp.abs(corr))
    amask = jnp.zeros((C, C), f32)
    for _ in range(TOPK):
        mrow = jnp.max(score, axis=1, keepdims=True)
        cand = jnp.where(score == mrow, iota_j, C)
        jmin = jnp.min(cand, axis=1, keepdims=True)
        hit = iota_j == jmin
        amask = jnp.where(hit, 1.0, amask)
        score = jnp.where(hit, score - 3.0, score)
    atmask = jnp.swapaxes(amask, 0, 1)

    xl = jnp.dot(nf, wl1_ref[...], preferred_element_type=f32) + bl1_ref[...]
    xr = jnp.dot(nf, wr1_ref[...], preferred_element_type=f32) + br1_ref[...]
    we1 = we1_ref[...]
    att1 = att1_ref[...]
    outs = []
    for h in range(HEADS):
        el = xl[:, GH * h : GH * (h + 1)]
        er = xr[:, GH * h : GH * (h + 1)]
        weh = we1[h : h + 1, :][None]
        ath = att1[h : h + 1, :][None]
        t3 = er[:, None, :] + el[None, :, :] + corr[:, :, None] * weh
        t3 = t3 * (0.6 * ath) + jnp.abs(t3) * (0.4 * ath)
        et = jnp.sum(t3, axis=-1)
        em = jnp.where(atmask > 0.5, et, -1e30)
        mj = jnp.max(em, axis=1, keepdims=True)
        mj = jnp.where(mj > -1e29, mj, 0.0)
        ex = jnp.where(atmask > 0.5, jnp.exp(et - mj), 0.0)
        sj = jnp.sum(ex, axis=1, keepdims=True)
        alpha = ex / (sj + 1e-16)
        outs.append(jnp.dot(alpha, el, preferred_element_type=f32))
    h1 = _elu(jnp.concatenate(outs, axis=1) + gb1_ref[...])

    xl2 = jnp.dot(h1, wl2_ref[...], preferred_element_type=f32) + bl2_ref[...]
    xr2 = jnp.dot(h1, wr2_ref[...], preferred_element_type=f32) + br2_ref[...]
    t3 = xr2[:, None, :] + xl2[None, :, :] + corr[:, :, None] * we2_ref[...][None]
    at2 = att2_ref[...][None]
    t3 = t3 * (0.6 * at2) + jnp.abs(t3) * (0.4 * at2)
    et = jnp.sum(t3, axis=-1)
    em = jnp.where(atmask > 0.5, et, -1e30)
    mj = jnp.max(em, axis=1, keepdims=True)
    mj = jnp.where(mj > -1e29, mj, 0.0)
    ex = jnp.where(atmask > 0.5, jnp.exp(et - mj), 0.0)
    sj = jnp.sum(ex, axis=1, keepdims=True)
    alpha = ex / (sj + 1e-16)
    h2 = _elu(jnp.dot(alpha, xl2, preferred_element_type=f32) + gb2_ref[...])

    g = jnp.sum(h2, axis=0, keepdims=True) / C
    a1v = jnp.tanh(jnp.dot(g, fc1w_ref[...], preferred_element_type=f32) + fc1b_ref[...])
    a2v = jnp.tanh(jnp.dot(g, fc2w_ref[...], preferred_element_type=f32) + fc2b_ref[...])
    gg = (
        g
        + (jnp.dot(a1v, p1w_ref[...], preferred_element_type=f32) + p1b_ref[...]) * a1p_ref[...]
        + (jnp.dot(a2v * a2v, p2w_ref[...], preferred_element_type=f32) + p2b_ref[...]) * a2p_ref[...]
    )
    gmu = jnp.mean(gg, axis=1, keepdims=True)
    gvar = jnp.mean((gg - gmu) * (gg - gmu), axis=1, keepdims=True)
    gn = (gg - gmu) / jnp.sqrt(gvar + 1e-5) * lng_ref[...] + lnb_ref[...]
    hh1 = jax.nn.relu(jnp.dot(gn, c1w_ref[...], preferred_element_type=f32) + c1b_ref[...])
    hh2 = jax.nn.relu(jnp.dot(hh1, c2w_ref[...], preferred_element_type=f32) + c2b_ref[...])
    res = jnp.dot(hh2, c3w_ref[...], preferred_element_type=f32) + c3b_ref[...]
    o_ref[...] = res[None]


def kernel(x, params):
    p = params
    f32 = _F32
    inv = 1.0 / jnp.sqrt(jnp.asarray(1.0 + 1e-5, f32))

    col = lambda v: v.reshape(-1, 1).astype(f32)
    row = lambda v: v.reshape(1, -1).astype(f32)

    dww = p['dw_w'].reshape(C, TK)
    s1 = col(p['bn1_g'] * inv)
    b1 = col(p['bn1_b'])
    spw = p['sp_w'].reshape(C * D)
    ss = p['bns_g'] * inv
    cse = col(spw[0::2] * ss[0::2])
    bse = col(p['bns_b'][0::2])
    cso = col(spw[1::2] * ss[1::2])
    bso = col(p['bns_b'][1::2])
    sdw = p['sd_w'].reshape(C * D, 16)
    sdwe = sdw[0::2]
    sdwo = sdw[1::2]
    sp2 = p['sp2_w'].reshape(C, C * D)
    sp2we = sp2[:, 0::2]
    sp2wo = sp2[:, 1::2]
    s2 = col(p['bn2_g'] * inv)
    b2 = col(p['bn2_b'])
    a3 = col(p['pw_w'].reshape(EMB) * (p['bn3_g'] * inv))
    b3 = col(p['bn3_b'])

    poolm = (jnp.arange(512)[:, None] // POOL == jnp.arange(128)[None, :]).astype(f32) / POOL

    wl1 = p['g1_Wl'].T
    wr1 = p['g1_Wr'].T
    we1 = p['g1_We'].reshape(HEADS, GH)
    att1 = p['g1_att'].reshape(HEADS, GH)
    wl2 = p['g2_Wl'].T
    wr2 = p['g2_Wr'].T
    we2 = p['g2_We'].reshape(1, GH)
    att2 = p['g2_att'].reshape(1, GH)

    operands = [
        dww, s1, b1,
        cse, bse, cso, bso,
        sdwe, sdwo,
        sp2we, sp2wo, s2, b2,
        a3, b3,
        poolm,
        wl1, row(p['g1_bl']), wr1, row(p['g1_br']), we1, att1, row(p['g1_bias']),
        wl2, row(p['g2_bl']), wr2, row(p['g2_br']), we2, att2, row(p['g2_bias']),
        p['fc1_w'].T, row(p['fc1_b']), p['fc2_w'].T, row(p['fc2_b']),
        p['p1_w'].T, row(p['p1_b']), p['p2_w'].T, row(p['p2_b']),
        row(p['a1']), row(p['a2']),
        row(p['ln_g']), row(p['ln_b']),
        p['c1_w'].T, row(p['c1_b']), p['c2_w'].T, row(p['c2_b']),
        p['c3_w'].T, row(p['c3_b']),
    ]

    wspecs = [
        pl.BlockSpec(o.shape, lambda b, _n=o.ndim: (0,) * _n) for o in operands
    ]
    return pl.pallas_call(
        _body,
        grid=(B,),
        in_specs=[pl.BlockSpec((1, C, T), lambda b: (b, 0, 0))] + wspecs,
        out_specs=pl.BlockSpec((1, 1, NC), lambda b: (b, 0, 0)),
        out_shape=jax.ShapeDtypeStruct((B, 1, NC), f32),
        compiler_params=pltpu.CompilerParams(
            dimension_semantics=("parallel",),
            vmem_limit_bytes=100 * 1024 * 1024,
        ),
    )(x.astype(f32), *operands).reshape(B, NC)


# speedup vs baseline: 2.7653x; 1.5193x over previous
import jax
import jax.numpy as jnp
from jax.experimental import pallas as pl
from jax.experimental.pallas import tpu as pltpu

B, C, T = 64, 64, 2048
EMB = 64
TK = 128
POOL = 4
TOPK = 8
GH = 128
HEADS = 8
NC = 2
D = 2

_F32 = jnp.float32


def _elu(v):
    p = v * (1.0 + v * (0.5 + v * (1.0 / 6.0 + v * (1.0 / 24.0 + v * (1.0 / 120.0)))))
    em1 = jnp.where(v > -0.25, p, jnp.exp(v) - 1.0)
    return jnp.where(v > 0, v, em1)


def _body(
    x_ref,
    dww_ref, s1_ref, b1_ref,
    cse_ref, bse_ref, cso_ref, bso_ref,
    sdwe_ref, sdwo_ref,
    sp2we_ref, sp2wo_ref, s2_ref, b2_ref,
    a3_ref, b3_ref,
    pm_ref,
    wl1_ref, bl1_ref, wr1_ref, br1_ref, we1r_ref, a06_ref, a04_ref,
    bsum_ref, bcast_ref, gb1_ref,
    wl2_ref, bl2_ref, wr2_ref, br2_ref, we2_ref, b06_ref, b04_ref,
    ones_gh_ref, gb2_ref,
    fc1w_ref, fc1b_ref, fc2w_ref, fc2b_ref,
    p1w_ref, p1b_ref, p2w_ref, p2b_ref, a1p_ref, a2p_ref,
    lng_ref, lnb_ref,
    c1w_ref, c1b_ref, c2w_ref, c2b_ref, c3w_ref, c3b_ref,
    o_ref,
):
    f32 = _F32
    xb = x_ref[0]

    zpad = jnp.zeros((C, TK // 2), f32)
    xpad = jnp.concatenate([zpad, xb, zpad[:, : TK // 2 - 1]], axis=1)
    dww = dww_ref[...]
    accs = [xpad[:, a : a + T] * dww[:, a : a + 1] for a in range(8)]
    for k in range(8, TK):
        a = k % 8
        accs[a] = accs[a] + xpad[:, k : k + T] * dww[:, k : k + 1]
    acc = ((accs[0] + accs[1]) + (accs[2] + accs[3])) + (
        (accs[4] + accs[5]) + (accs[6] + accs[7])
    )
    y1 = _elu(acc * s1_ref[...] + b1_ref[...])

    ye = _elu(y1 * cse_ref[...] + bse_ref[...])
    yo = _elu(y1 * cso_ref[...] + bso_ref[...])
    pm = pm_ref[...]
    pe = jnp.concatenate(
        [jnp.dot(ye[:, 512 * m : 512 * (m + 1)], pm, preferred_element_type=f32)
         for m in range(4)], axis=1)
    po = jnp.concatenate(
        [jnp.dot(yo[:, 512 * m : 512 * (m + 1)], pm, preferred_element_type=f32)
         for m in range(4)], axis=1)

    TP = T // POOL
    z8 = jnp.zeros((C, 8), f32)
    pep = jnp.concatenate([z8, pe, z8[:, :7]], axis=1)
    pop = jnp.concatenate([z8, po, z8[:, :7]], axis=1)
    sdwe = sdwe_ref[...]
    sdwo = sdwo_ref[...]
    ses = [pep[:, a : a + TP] * sdwe[:, a : a + 1] for a in range(4)]
    sos = [pop[:, a : a + TP] * sdwo[:, a : a + 1] for a in range(4)]
    for k in range(4, 16):
        a = k % 4
        ses[a] = ses[a] + pep[:, k : k + TP] * sdwe[:, k : k + 1]
        sos[a] = sos[a] + pop[:, k : k + TP] * sdwo[:, k : k + 1]
    se = (ses[0] + ses[1]) + (ses[2] + ses[3])
    so = (sos[0] + sos[1]) + (sos[2] + sos[3])

    z0 = jnp.dot(sp2we_ref[...], se, preferred_element_type=f32) + jnp.dot(
        sp2wo_ref[...], so, preferred_element_type=f32
    )
    zn = _elu(z0 * s2_ref[...] + b2_ref[...])
    zz = jnp.dot(zn, pm, preferred_element_type=f32)

    v3 = _elu(zz[:, None, :] * a3_ref[...][None] + b3_ref[...][None])
    nf = jnp.mean(v3, axis=-1)

    mu = jnp.mean(xb, axis=1, keepdims=True)
    xm = xb - mu
    var1 = jnp.sum(xm * xm, axis=1, keepdims=True) / (T - 1)
    xs = xm / (jnp.sqrt(var1) + 1e-8)
    corr = jax.lax.dot_general(
        xs, xs, (((1,), (1,)), ((), ())), preferred_element_type=f32
    ) / (T - 1)
    corr = jnp.clip(corr, -1.0, 1.0)

    iota_i = jax.lax.broadcasted_iota(jnp.int32, (C, C), 0)
    iota_j = jax.lax.broadcasted_iota(jnp.int32, (C, C), 1)
    score = jnp.where(iota_i == iota_j, -1.0, jnp.abs(corr))
    hits = []
    hitst = []
    wvals = []
    for _ in range(TOPK):
        mrow = jnp.max(score, axis=1, keepdims=True)
        cand = jnp.where(score == mrow, iota_j, C)
        jmin = jnp.min(cand, axis=1, keepdims=True)
        hit = iota_j == jmin
        hits.append(hit.astype(f32))
        hitst.append((iota_i == jnp.swapaxes(jmin, 0, 1)).astype(f32))
        wvals.append(jnp.sum(jnp.where(hit, corr, 0.0), axis=1, keepdims=True))
        score = jnp.where(hit, score - 3.0, score)

    xl = jnp.dot(nf, wl1_ref[...], preferred_element_type=f32) + bl1_ref[...]
    xr = jnp.dot(nf, wr1_ref[...], preferred_element_type=f32) + br1_ref[...]
    we1r = we1r_ref[...]
    a06 = a06_ref[...]
    a04 = a04_ref[...]
    bsum = bsum_ref[...]
    e8s = []
    for r in range(TOPK):
        er_sel = jnp.dot(hits[r], xr, preferred_element_type=f32)
        s = xl + er_sel + wvals[r] * we1r
        u = s * a06 + jnp.abs(s) * a04
        e8s.append(jnp.dot(u, bsum, preferred_element_type=f32))
    mg = e8s[0]
    for r in range(1, TOPK):
        mg = jnp.maximum(mg, e8s[r])
    mg = jnp.max(mg, axis=0, keepdims=True)
    exs = []
    sseg = jnp.zeros((C, HEADS), f32)
    for r in range(TOPK):
        ex = jnp.exp(e8s[r] - mg)
        exs.append(ex)
        sseg = sseg + jnp.dot(hitst[r], ex, preferred_element_type=f32)
    bcast = bcast_ref[...]
    out1 = jnp.zeros((C, HEADS * GH), f32)
    for r in range(TOPK):
        den = jnp.dot(hits[r], sseg, preferred_element_type=f32) + 1e-16
        a1024 = jnp.dot(exs[r] / den, bcast, preferred_element_type=f32)
        out1 = out1 + jnp.dot(hitst[r], xl * a1024, preferred_element_type=f32)
    h1 = _elu(out1 + gb1_ref[...])

    xl2 = jnp.dot(h1, wl2_ref[...], preferred_element_type=f32) + bl2_ref[...]
    xr2 = jnp.dot(h1, wr2_ref[...], preferred_element_type=f32) + br2_ref[...]
    we2r = we2_ref[...]
    b06 = b06_ref[...]
    b04 = b04_ref[...]
    ones_gh = ones_gh_ref[...]
    e1s = []
    for r in range(TOPK):
        er_sel = jnp.dot(hits[r], xr2, preferred_element_type=f32)
        s = xl2 + er_sel + wvals[r] * we2r
        u = s * b06 + jnp.abs(s) * b04
        e1s.append(jnp.dot(u, ones_gh, preferred_element_type=f32))
    mg2 = e1s[0]
    for r in range(1, TOPK):
        mg2 = jnp.maximum(mg2, e1s[r])
    mg2 = jnp.max(mg2, axis=0, keepdims=True)
    exs2 = []
    sseg2 = jnp.zeros((C, 1), f32)
    for r in range(TOPK):
        ex = jnp.exp(e1s[r] - mg2)
        exs2.append(ex)
        sseg2 = sseg2 + jnp.dot(hitst[r], ex, preferred_element_type=f32)
    out2 = jnp.zeros((C, GH), f32)
    for r in range(TOPK):
        den = jnp.dot(hits[r], sseg2, preferred_element_type=f32) + 1e-16
        out2 = out2 + jnp.dot(hitst[r], xl2 * (exs2[r] / den), preferred_element_type=f32)
    h2 = _elu(out2 + gb2_ref[...])

    g = jnp.sum(h2, axis=0, keepdims=True) / C
    a1v = jnp.tanh(jnp.dot(g, fc1w_ref[...], preferred_element_type=f32) + fc1b_ref[...])
    a2v = jnp.tanh(jnp.dot(g, fc2w_ref[...], preferred_element_type=f32) + fc2b_ref[...])
    gg = (
        g
        + (jnp.dot(a1v, p1w_ref[...], preferred_element_type=f32) + p1b_ref[...]) * a1p_ref[...]
        + (jnp.dot(a2v * a2v, p2w_ref[...], preferred_element_type=f32) + p2b_ref[...]) * a2p_ref[...]
    )
    gmu = jnp.mean(gg, axis=1, keepdims=True)
    gvar = jnp.mean((gg - gmu) * (gg - gmu), axis=1, keepdims=True)
    gn = (gg - gmu) / jnp.sqrt(gvar + 1e-5) * lng_ref[...] + lnb_ref[...]
    hh1 = jax.nn.relu(jnp.dot(gn, c1w_ref[...], preferred_element_type=f32) + c1b_ref[...])
    hh2 = jax.nn.relu(jnp.dot(hh1, c2w_ref[...], preferred_element_type=f32) + c2b_ref[...])
    res = jnp.dot(hh2, c3w_ref[...], preferred_element_type=f32) + c3b_ref[...]
    o_ref[...] = res[None]


def kernel(x, params):
    p = params
    f32 = _F32
    inv = 1.0 / jnp.sqrt(jnp.asarray(1.0 + 1e-5, f32))

    col = lambda v: v.reshape(-1, 1).astype(f32)
    row = lambda v: v.reshape(1, -1).astype(f32)

    dww = p['dw_w'].reshape(C, TK)
    s1 = col(p['bn1_g'] * inv)
    b1 = col(p['bn1_b'])
    spw = p['sp_w'].reshape(C * D)
    ss = p['bns_g'] * inv
    cse = col(spw[0::2] * ss[0::2])
    bse = col(p['bns_b'][0::2])
    cso = col(spw[1::2] * ss[1::2])
    bso = col(p['bns_b'][1::2])
    sdw = p['sd_w'].reshape(C * D, 16)
    sdwe = sdw[0::2]
    sdwo = sdw[1::2]
    sp2 = p['sp2_w'].reshape(C, C * D)
    sp2we = sp2[:, 0::2]
    sp2wo = sp2[:, 1::2]
    s2 = col(p['bn2_g'] * inv)
    b2 = col(p['bn2_b'])
    a3 = col(p['pw_w'].reshape(EMB) * (p['bn3_g'] * inv))
    b3 = col(p['bn3_b'])

    poolm = (jnp.arange(512)[:, None] // POOL == jnp.arange(128)[None, :]).astype(f32) / POOL

    wl1 = p['g1_Wl'].T
    wr1 = p['g1_Wr'].T
    we1r = p['g1_We'].reshape(1, HEADS * GH)
    att1024 = p['g1_att'].reshape(1, HEADS * GH)
    bsum = (jnp.arange(HEADS * GH)[:, None] // GH == jnp.arange(HEADS)[None, :]).astype(f32)
    wl2 = p['g2_Wl'].T
    wr2 = p['g2_Wr'].T
    we2 = p['g2_We'].reshape(1, GH)
    att2 = p['g2_att'].reshape(1, GH)
    ones_gh = jnp.ones((GH, 1), f32)

    operands = [
        dww, s1, b1,
        cse, bse, cso, bso,
        sdwe, sdwo,
        sp2we, sp2wo, s2, b2,
        a3, b3,
        poolm,
        wl1, row(p['g1_bl']), wr1, row(p['g1_br']), we1r, 0.6 * att1024, 0.4 * att1024,
        bsum, bsum.T, row(p['g1_bias']),
        wl2, row(p['g2_bl']), wr2, row(p['g2_br']), we2, 0.6 * att2, 0.4 * att2,
        ones_gh, row(p['g2_bias']),
        p['fc1_w'].T, row(p['fc1_b']), p['fc2_w'].T, row(p['fc2_b']),
        p['p1_w'].T, row(p['p1_b']), p['p2_w'].T, row(p['p2_b']),
        row(p['a1']), row(p['a2']),
        row(p['ln_g']), row(p['ln_b']),
        p['c1_w'].T, row(p['c1_b']), p['c2_w'].T, row(p['c2_b']),
        p['c3_w'].T, row(p['c3_b']),
    ]

    wspecs = [
        pl.BlockSpec(o.shape, lambda b, _n=o.ndim: (0,) * _n) for o in operands
    ]
    return pl.pallas_call(
        _body,
        grid=(B,),
        in_specs=[pl.BlockSpec((1, C, T), lambda b: (b, 0, 0))] + wspecs,
        out_specs=pl.BlockSpec((1, 1, NC), lambda b: (b, 0, 0)),
        out_shape=jax.ShapeDtypeStruct((B, 1, NC), f32),
        compiler_params=pltpu.CompilerParams(
            dimension_semantics=("parallel",),
            vmem_limit_bytes=100 * 1024 * 1024,
        ),
    )(x.astype(f32), *operands).reshape(B, NC)
